# Initial kernel scaffold; baseline (speedup 1.0000x reference)
import jax
import jax.numpy as jnp
from jax.experimental import pallas as pl


def kernel(x, W_in, b_in, W_experts, W_gate, W_t, b_t, W_out, b_out):
    raise NotImplementedError("write your pallas kernel here")


# speedup vs baseline: 1.4889x; 1.4889x over previous
import jax
import jax.numpy as jnp
from jax.experimental import pallas as pl
from jax.experimental.pallas import tpu as pltpu

B = 2048
H = 1024
E = 8
IN = 3072
OUT = 10
NEG_INF = -1e30
BF = jnp.bfloat16


def _bdot(a, b):
    return jnp.dot(a, b, preferred_element_type=jnp.float32)


def _in_kernel(x_ref, w_ref, b_ref, o_ref):
    acc = _bdot(x_ref[...], w_ref[...])
    o_ref[...] = jax.nn.relu(acc + b_ref[...])


def _gate_kernel(h_ref, wg_ref, logits_ref, comb_ref):
    l = _bdot(h_ref[...].astype(BF), wg_ref[...])
    logits_ref[...] = l
    iota = jax.lax.broadcasted_iota(jnp.int32, l.shape, 1)
    v1 = jnp.max(l, axis=-1, keepdims=True)
    i1 = jnp.min(jnp.where(l == v1, iota, E), axis=-1, keepdims=True)
    l2 = jnp.where(iota == i1, NEG_INF, l)
    v2 = jnp.max(l2, axis=-1, keepdims=True)
    i2 = jnp.min(jnp.where(l2 == v2, iota, E), axis=-1, keepdims=True)
    w1 = 1.0 / (1.0 + jnp.exp(v2 - v1))
    w2 = 1.0 - w1
    comb_ref[...] = jnp.where(iota == i1, w1, 0.0) + jnp.where(iota == i2, w2, 0.0)


def _moe_kernel(comb_ref, h_ref, we_ref, o_ref):
    e = pl.program_id(1)
    y = jax.nn.relu(_bdot(h_ref[...].astype(BF), we_ref[0]))
    lane = jax.lax.broadcasted_iota(jnp.int32, comb_ref.shape, 1)
    w = jnp.sum(jnp.where(lane == e, comb_ref[...], 0.0), axis=-1, keepdims=True)
    contrib = w.astype(BF).astype(jnp.float32) * y.astype(BF).astype(jnp.float32)

    @pl.when(e == 0)
    def _():
        o_ref[...] = contrib

    @pl.when(e > 0)
    def _():
        o_ref[...] += contrib


def _trans_kernel(h_ref, w_ref, b_ref, o_ref):
    acc = _bdot(h_ref[...].astype(BF), w_ref[...])
    o_ref[...] = acc + b_ref[...]


def _input_layer(x2d, W_in, b_in):
    TM = 512
    return pl.pallas_call(
        _in_kernel,
        grid=(B // TM,),
        in_specs=[
            pl.BlockSpec((TM, IN), lambda m: (m, 0)),
            pl.BlockSpec((IN, H), lambda m: (0, 0)),
            pl.BlockSpec((1, H), lambda m: (0, 0)),
        ],
        out_specs=pl.BlockSpec((TM, H), lambda m: (m, 0)),
        out_shape=jax.ShapeDtypeStruct((B, H), jnp.float32),
    )(x2d, W_in, b_in)


def _gate(h, W_gate):
    return pl.pallas_call(
        _gate_kernel,
        grid=(1,),
        in_specs=[
            pl.BlockSpec((B, H), lambda m: (0, 0)),
            pl.BlockSpec((H, E), lambda m: (0, 0)),
        ],
        out_specs=[
            pl.BlockSpec((B, E), lambda m: (0, 0)),
            pl.BlockSpec((B, E), lambda m: (0, 0)),
        ],
        out_shape=[
            jax.ShapeDtypeStruct((B, E), jnp.float32),
            jax.ShapeDtypeStruct((B, E), jnp.float32),
        ],
    )(h, W_gate)


def _moe(comb, h, W_experts):
    TM = 1024
    return pl.pallas_call(
        _moe_kernel,
        grid=(B // TM, E),
        in_specs=[
            pl.BlockSpec((TM, E), lambda m, e: (m, 0)),
            pl.BlockSpec((TM, H), lambda m, e: (m, 0)),
            pl.BlockSpec((1, H, H), lambda m, e: (e, 0, 0)),
        ],
        out_specs=pl.BlockSpec((TM, H), lambda m, e: (m, 0)),
        out_shape=jax.ShapeDtypeStruct((B, H), jnp.float32),
        compiler_params=pltpu.CompilerParams(
            dimension_semantics=("parallel", "arbitrary")),
    )(comb, h, W_experts)


def _trans(h, W_t, b_t, n_cols):
    TM = 1024
    return pl.pallas_call(
        _trans_kernel,
        grid=(B // TM,),
        in_specs=[
            pl.BlockSpec((TM, H), lambda m: (m, 0)),
            pl.BlockSpec((H, n_cols), lambda m: (0, 0)),
            pl.BlockSpec((1, n_cols), lambda m: (0, 0)),
        ],
        out_specs=pl.BlockSpec((TM, n_cols), lambda m: (m, 0)),
        out_shape=jax.ShapeDtypeStruct((B, n_cols), jnp.float32),
    )(h, W_t, b_t)


def kernel(x, W_in, b_in, W_experts, W_gate, W_t, b_t, W_out, b_out):
    x2d = x.reshape(x.shape[0], -1).astype(BF)
    W_in = W_in.astype(BF)
    W_experts = W_experts.astype(BF)
    W_gate = W_gate.astype(BF)
    W_t = W_t.astype(BF)
    W_out = W_out.astype(BF)
    h = _input_layer(x2d, W_in, b_in.reshape(1, H))
    logits_list = []
    for _ in range(2):
        logits, comb = _gate(h, W_gate)
        logits_list.append(logits)
        h = _moe(comb, h, W_experts)
        h = _trans(h, W_t, b_t.reshape(1, H), H)
    out = _trans(h, W_out, b_out.reshape(1, OUT), OUT)
    return (out,) + tuple(logits_list)


# speedup vs baseline: 1.6858x; 1.1322x over previous
import jax
import jax.numpy as jnp
from jax.experimental import pallas as pl
from jax.experimental.pallas import tpu as pltpu

B = 2048
H = 1024
E = 8
IN = 3072
OUT = 10
NEG_INF = -1e30
BF = jnp.bfloat16
F32 = jnp.float32


def _bdot(a, b):
    return jnp.dot(a, b, preferred_element_type=F32)


def _in_kernel(x_ref, w_ref, b_ref, o_ref):
    acc = _bdot(x_ref[...], w_ref[...])
    o_ref[...] = jax.nn.relu(acc + b_ref[...]).astype(BF)


def _gate_comb(h, wg):
    l = _bdot(h, wg)
    iota = jax.lax.broadcasted_iota(jnp.int32, l.shape, 1)
    v1 = jnp.max(l, axis=-1, keepdims=True)
    i1 = jnp.min(jnp.where(l == v1, iota, E), axis=-1, keepdims=True)
    l2 = jnp.where(iota == i1, NEG_INF, l)
    v2 = jnp.max(l2, axis=-1, keepdims=True)
    i2 = jnp.min(jnp.where(l2 == v2, iota, E), axis=-1, keepdims=True)
    w1 = 1.0 / (1.0 + jnp.exp(v2 - v1))
    w2 = 1.0 - w1
    comb = jnp.where(iota == i1, w1, 0.0) + jnp.where(iota == i2, w2, 0.0)
    return l, comb


def _round_kernel(h_ref, wg_ref, we_ref, wt_ref, bt_ref,
                  logits_ref, h1_ref, comb_ref, acc_ref):
    e = pl.program_id(0)

    @pl.when(e == 0)
    def _():
        l, comb = _gate_comb(h_ref[...], wg_ref[...])
        logits_ref[...] = l
        comb_ref[...] = comb

    y = jax.nn.relu(_bdot(h_ref[...], we_ref[0]))
    lane = jax.lax.broadcasted_iota(jnp.int32, comb_ref.shape, 1)
    w = jnp.sum(jnp.where(lane == e, comb_ref[...], 0.0), axis=-1, keepdims=True)
    contrib = w.astype(BF).astype(F32) * y.astype(BF).astype(F32)

    @pl.when(e == 0)
    def _():
        acc_ref[...] = contrib

    @pl.when(e > 0)
    def _():
        acc_ref[...] += contrib

    @pl.when(e == E - 1)
    def _():
        z = _bdot(acc_ref[...].astype(BF), wt_ref[...]) + bt_ref[...]
        h1_ref[...] = z.astype(BF)


def _round2_kernel(h_ref, wg_ref, we_ref, wt_ref, bt_ref, wo_ref, bo_ref,
                   logits_ref, out_ref, comb_ref, acc_ref):
    e = pl.program_id(0)

    @pl.when(e == 0)
    def _():
        l, comb = _gate_comb(h_ref[...], wg_ref[...])
        logits_ref[...] = l
        comb_ref[...] = comb

    y = jax.nn.relu(_bdot(h_ref[...], we_ref[0]))
    lane = jax.lax.broadcasted_iota(jnp.int32, comb_ref.shape, 1)
    w = jnp.sum(jnp.where(lane == e, comb_ref[...], 0.0), axis=-1, keepdims=True)
    contrib = w.astype(BF).astype(F32) * y.astype(BF).astype(F32)

    @pl.when(e == 0)
    def _():
        acc_ref[...] = contrib

    @pl.when(e > 0)
    def _():
        acc_ref[...] += contrib

    @pl.when(e == E - 1)
    def _():
        z = _bdot(acc_ref[...].astype(BF), wt_ref[...]) + bt_ref[...]
        out_ref[...] = _bdot(z.astype(BF), wo_ref[...]) + bo_ref[...]


def _input_layer(x2d, W_in, b_in):
    TM = 1024
    return pl.pallas_call(
        _in_kernel,
        grid=(B // TM,),
        in_specs=[
            pl.BlockSpec((TM, IN), lambda m: (m, 0)),
            pl.BlockSpec((IN, H), lambda m: (0, 0)),
            pl.BlockSpec((1, H), lambda m: (0, 0)),
        ],
        out_specs=pl.BlockSpec((TM, H), lambda m: (m, 0)),
        out_shape=jax.ShapeDtypeStruct((B, H), BF),
    )(x2d, W_in, b_in)


def _round(h, W_gate, W_experts, W_t, b_t):
    return pl.pallas_call(
        _round_kernel,
        grid=(E,),
        in_specs=[
            pl.BlockSpec((B, H), lambda e: (0, 0)),
            pl.BlockSpec((H, E), lambda e: (0, 0)),
            pl.BlockSpec((1, H, H), lambda e: (e, 0, 0)),
            pl.BlockSpec((H, H), lambda e: (0, 0)),
            pl.BlockSpec((1, H), lambda e: (0, 0)),
        ],
        out_specs=[
            pl.BlockSpec((B, E), lambda e: (0, 0)),
            pl.BlockSpec((B, H), lambda e: (0, 0)),
        ],
        out_shape=[
            jax.ShapeDtypeStruct((B, E), F32),
            jax.ShapeDtypeStruct((B, H), BF),
        ],
        scratch_shapes=[
            pltpu.VMEM((B, E), F32),
            pltpu.VMEM((B, H), F32),
        ],
        compiler_params=pltpu.CompilerParams(
            dimension_semantics=("arbitrary",)),
    )(h, W_gate, W_experts, W_t, b_t)


def _round2(h, W_gate, W_experts, W_t, b_t, W_out, b_out):
    return pl.pallas_call(
        _round2_kernel,
        grid=(E,),
        in_specs=[
            pl.BlockSpec((B, H), lambda e: (0, 0)),
            pl.BlockSpec((H, E), lambda e: (0, 0)),
            pl.BlockSpec((1, H, H), lambda e: (e, 0, 0)),
            pl.BlockSpec((H, H), lambda e: (0, 0)),
            pl.BlockSpec((1, H), lambda e: (0, 0)),
            pl.BlockSpec((H, OUT), lambda e: (0, 0)),
            pl.BlockSpec((1, OUT), lambda e: (0, 0)),
        ],
        out_specs=[
            pl.BlockSpec((B, E), lambda e: (0, 0)),
            pl.BlockSpec((B, OUT), lambda e: (0, 0)),
        ],
        out_shape=[
            jax.ShapeDtypeStruct((B, E), F32),
            jax.ShapeDtypeStruct((B, OUT), F32),
        ],
        scratch_shapes=[
            pltpu.VMEM((B, E), F32),
            pltpu.VMEM((B, H), F32),
        ],
        compiler_params=pltpu.CompilerParams(
            dimension_semantics=("arbitrary",)),
    )(h, W_gate, W_experts, W_t, b_t, W_out, b_out)


def kernel(x, W_in, b_in, W_experts, W_gate, W_t, b_t, W_out, b_out):
    x2d = x.reshape(x.shape[0], -1).astype(BF)
    h = _input_layer(x2d, W_in.astype(BF), b_in.reshape(1, H))
    logits1, h = _round(h, W_gate.astype(BF), W_experts.astype(BF),
                        W_t.astype(BF), b_t.reshape(1, H))
    logits2, out = _round2(h, W_gate.astype(BF), W_experts.astype(BF),
                           W_t.astype(BF), b_t.reshape(1, H),
                           W_out.astype(BF), b_out.reshape(1, OUT))
    return (out, logits1, logits2)
